# baseline (speedup 1.0000x reference)
import jax
import jax.numpy as jnp
from jax.experimental import pallas as pl
from jax.experimental.pallas import tpu as pltpu

_CROP_TOP = 40


def _make_body(n_valid):
    def _mix_body(x_ref, m_ref, we_ref, be_ref, wd_ref, bd_ref, o_ref):
        t = pl.program_id(0)

        @pl.when(t < n_valid)
        def _compute():
            h = jnp.dot(we_ref[...], x_ref[...],
                        preferred_element_type=jnp.float32)
            h = jnp.maximum(h + be_ref[...], 0.0)
            d = jnp.dot(wd_ref[...], h, preferred_element_type=jnp.float32)
            o_ref[...] = (d + bd_ref[...]) * m_ref[...]

        @pl.when(t >= n_valid)
        def _zero():
            o_ref[...] = jnp.zeros_like(o_ref)

    return _mix_body


def kernel(X, mask2d, g_w, g_b, ae_w1, ae_b1, ae_w2, ae_b2):
    N, C, H, W = X.shape
    Hd = ae_w1.shape[1]
    NC, HW = N * C, H * W
    NHd = N * Hd

    hp = jax.lax.Precision.HIGHEST
    w_enc = jnp.dot(ae_w1.T, g_w.T, precision=hp)
    b_enc = (jnp.dot(ae_w1.T, g_b.reshape(-1, 1), precision=hp)
             + ae_b1.reshape(-1, 1))
    we_blk = jax.scipy.linalg.block_diag(*([w_enc] * N))
    be_blk = jnp.tile(b_enc, (N, 1))
    wd_blk = jax.scipy.linalg.block_diag(*([ae_w2.T] * N))
    bd_blk = jnp.tile(ae_b2.reshape(-1, 1), (N, 1))

    tp = 8 * W
    n_tiles = HW // tp
    valid_pix = (H - _CROP_TOP) * W
    n_valid = valid_pix // tp

    Xf = X.reshape(NC, HW)
    mf = mask2d.reshape(1, HW).astype(jnp.float32)

    out = pl.pallas_call(
        _make_body(n_valid),
        grid=(n_tiles,),
        in_specs=[
            pl.BlockSpec((NC, tp), lambda t: (0, t)),
            pl.BlockSpec((1, tp), lambda t: (0, t)),
            pl.BlockSpec((NHd, NC), lambda t: (0, 0)),
            pl.BlockSpec((NHd, 1), lambda t: (0, 0)),
            pl.BlockSpec((NC, NHd), lambda t: (0, 0)),
            pl.BlockSpec((NC, 1), lambda t: (0, 0)),
        ],
        out_specs=pl.BlockSpec((NC, tp), lambda t: (0, t)),
        out_shape=jax.ShapeDtypeStruct((NC, HW), jnp.float32),
        compiler_params=pltpu.CompilerParams(
            dimension_semantics=("parallel",)),
    )(Xf, mf, we_blk, be_blk, wd_blk, bd_blk)
    return out.reshape(N, C, H, W)


# speedup vs baseline: 1.8659x; 1.8659x over previous
import jax
import jax.numpy as jnp
from jax.experimental import pallas as pl
from jax.experimental.pallas import tpu as pltpu

_CROP_TOP = 40


def _make_body(C, R, S, RB, W, valid_rows):
    def _body(x_ref, m_ref, we_ref, be_ref, wd_ref, bd_ref, o_ref):
        t = pl.program_id(1)
        rows = t * RB + jax.lax.broadcasted_iota(jnp.int32, (RB, W), 0)
        m = m_ref[...] * (rows < valid_rows).astype(jnp.float32)
        m4 = m.reshape(S, R, W)
        x4 = x_ref[0].reshape(C, S, R, W)
        we = we_ref[...]
        be = be_ref[...]
        wd = wd_ref[...]
        bd = bd_ref[...]
        outs = []
        for s in range(S):
            xs = x4[:, s].reshape(C * R, W)
            h = jnp.dot(we, xs, preferred_element_type=jnp.float32)
            h = jnp.maximum(h + be, 0.0)
            d = jnp.dot(wd, h, preferred_element_type=jnp.float32)
            outs.append((d + bd).reshape(C, R, W) * m4[s][None])
        o_ref[...] = jnp.stack(outs, axis=1).reshape(1, C, RB, W)

    return _body


def kernel(X, mask2d, g_w, g_b, ae_w1, ae_b1, ae_w2, ae_b2):
    N, C, H, W = X.shape
    Hd = ae_w1.shape[1]

    hp = jax.lax.Precision.HIGHEST
    w_enc = jnp.dot(ae_w1.T, g_w.T, precision=hp)
    b_enc = (jnp.dot(ae_w1.T, g_b.reshape(-1, 1), precision=hp)
             + ae_b1.reshape(-1, 1))

    R = 16
    eye = jnp.eye(R, dtype=jnp.float32)
    we_k = jnp.kron(w_enc, eye)
    wd_k = jnp.kron(ae_w2.T, eye)
    be_k = jnp.repeat(b_enc.reshape(-1), R).reshape(-1, 1)
    bd_k = jnp.repeat(ae_b2.reshape(-1), R).reshape(-1, 1)

    RB = 256
    while H % RB or RB % R:
        RB //= 2
    S = RB // R
    T = H // RB
    valid_rows = H - _CROP_TOP

    out = pl.pallas_call(
        _make_body(C, R, S, RB, W, valid_rows),
        grid=(N, T),
        in_specs=[
            pl.BlockSpec((1, C, RB, W), lambda n, t: (n, 0, t, 0)),
            pl.BlockSpec((RB, W), lambda n, t: (t, 0)),
            pl.BlockSpec((Hd * R, C * R), lambda n, t: (0, 0)),
            pl.BlockSpec((Hd * R, 1), lambda n, t: (0, 0)),
            pl.BlockSpec((C * R, Hd * R), lambda n, t: (0, 0)),
            pl.BlockSpec((C * R, 1), lambda n, t: (0, 0)),
        ],
        out_specs=pl.BlockSpec((1, C, RB, W), lambda n, t: (n, 0, t, 0)),
        out_shape=jax.ShapeDtypeStruct((N, C, H, W), jnp.float32),
        compiler_params=pltpu.CompilerParams(
            dimension_semantics=("parallel", "arbitrary")),
    )(X, mask2d.astype(jnp.float32), we_k, be_k, wd_k, bd_k)
    return out


# speedup vs baseline: 2.9165x; 1.5631x over previous
import jax
import jax.numpy as jnp
from jax.experimental import pallas as pl
from jax.experimental.pallas import tpu as pltpu

_CROP_TOP = 40


def _make_body(C, R, S, RB, W, valid_rows):
    def _body(x_ref, m_ref, we_ref, be_ref, wd_ref, bd_ref, o_ref):
        t = pl.program_id(1)
        rows = t * RB + jax.lax.broadcasted_iota(jnp.int32, (RB, W), 0)
        m = m_ref[...] * (rows < valid_rows).astype(jnp.float32)
        m4 = m.reshape(S, R, W)
        x4 = x_ref[0].reshape(C, S, R, W)
        we = we_ref[...]
        be = be_ref[...]
        wd = wd_ref[...]
        bd = bd_ref[...]
        hs = []
        for s in range(S):
            xs = x4[:, s].reshape(C * R, W)
            h = jnp.dot(we, xs, preferred_element_type=jnp.float32)
            hs.append(jnp.maximum(h + be, 0.0))
        outs = []
        for s in range(S):
            d = jnp.dot(wd, hs[s], preferred_element_type=jnp.float32)
            outs.append((d + bd).reshape(C, R, W) * m4[s][None])
        o_ref[...] = jnp.stack(outs, axis=1).reshape(1, C, RB, W)

    return _body


def kernel(X, mask2d, g_w, g_b, ae_w1, ae_b1, ae_w2, ae_b2):
    N, C, H, W = X.shape
    Hd = ae_w1.shape[1]

    hp = jax.lax.Precision.HIGHEST
    w_enc = jnp.dot(ae_w1.T, g_w.T, precision=hp)
    b_enc = (jnp.dot(ae_w1.T, g_b.reshape(-1, 1), precision=hp)
             + ae_b1.reshape(-1, 1))

    R = 16
    eye = jnp.eye(R, dtype=jnp.float32)
    we_k = jnp.kron(w_enc, eye)
    wd_k = jnp.kron(ae_w2.T, eye)
    be_k = jnp.repeat(b_enc.reshape(-1), R).reshape(-1, 1)
    bd_k = jnp.repeat(ae_b2.reshape(-1), R).reshape(-1, 1)

    RB = 256
    while H % RB or RB % R:
        RB //= 2
    S = RB // R
    T = H // RB
    valid_rows = H - _CROP_TOP

    out = pl.pallas_call(
        _make_body(C, R, S, RB, W, valid_rows),
        grid=(N, T),
        in_specs=[
            pl.BlockSpec((1, C, RB, W), lambda n, t: (n, 0, t, 0)),
            pl.BlockSpec((RB, W), lambda n, t: (t, 0)),
            pl.BlockSpec((Hd * R, C * R), lambda n, t: (0, 0)),
            pl.BlockSpec((Hd * R, 1), lambda n, t: (0, 0)),
            pl.BlockSpec((C * R, Hd * R), lambda n, t: (0, 0)),
            pl.BlockSpec((C * R, 1), lambda n, t: (0, 0)),
        ],
        out_specs=pl.BlockSpec((1, C, RB, W), lambda n, t: (n, 0, t, 0)),
        out_shape=jax.ShapeDtypeStruct((N, C, H, W), jnp.float32),
        compiler_params=pltpu.CompilerParams(
            dimension_semantics=("parallel", "arbitrary")),
    )(X, mask2d.astype(jnp.float32), we_k, be_k, wd_k, bd_k)
    return out


# speedup vs baseline: 3.0084x; 1.0315x over previous
import jax
import jax.numpy as jnp
from jax.experimental import pallas as pl
from jax.experimental.pallas import tpu as pltpu

_CROP_TOP = 40


def _make_body(C, R, S, RB, W, valid_rows):
    def _body(x_ref, m_ref, we_ref, be_ref, wd_ref, bd_ref, o_ref):
        t = pl.program_id(0)
        rows = t * RB + jax.lax.broadcasted_iota(jnp.int32, (RB, W), 0)
        m = m_ref[...] * (rows < valid_rows).astype(jnp.float32)
        m4 = m.reshape(S, R, W)
        x4 = x_ref[0].reshape(C, S, R, W)
        we = we_ref[...]
        be = be_ref[...]
        wd = wd_ref[...]
        bd = bd_ref[...]
        hs = []
        for s in range(S):
            xs = x4[:, s].reshape(C * R, W).astype(jnp.bfloat16)
            h = jnp.dot(we, xs, preferred_element_type=jnp.float32)
            hs.append(jnp.maximum(h + be, 0.0).astype(jnp.bfloat16))
        for s in range(S):
            d = jnp.dot(wd, hs[s], preferred_element_type=jnp.float32)
            o_ref[0, :, s * R:(s + 1) * R, :] = (
                (d + bd).reshape(C, R, W) * m4[s][None])

    return _body


def kernel(X, mask2d, g_w, g_b, ae_w1, ae_b1, ae_w2, ae_b2):
    N, C, H, W = X.shape
    Hd = ae_w1.shape[1]

    hp = jax.lax.Precision.HIGHEST
    w_enc = jnp.dot(ae_w1.T, g_w.T, precision=hp)
    b_enc = (jnp.dot(ae_w1.T, g_b.reshape(-1, 1), precision=hp)
             + ae_b1.reshape(-1, 1))

    R = 16
    eye = jnp.eye(R, dtype=jnp.float32)
    we_k = jnp.kron(w_enc, eye).astype(jnp.bfloat16)
    wd_k = jnp.kron(ae_w2.T, eye).astype(jnp.bfloat16)
    be_k = jnp.repeat(b_enc.reshape(-1), R).reshape(-1, 1)
    bd_k = jnp.repeat(ae_b2.reshape(-1), R).reshape(-1, 1)

    RB = 256
    while H % RB or RB % R:
        RB //= 2
    S = RB // R
    T = H // RB
    valid_rows = H - _CROP_TOP

    out = pl.pallas_call(
        _make_body(C, R, S, RB, W, valid_rows),
        grid=(T, N),
        in_specs=[
            pl.BlockSpec((1, C, RB, W), lambda t, n: (n, 0, t, 0)),
            pl.BlockSpec((RB, W), lambda t, n: (t, 0)),
            pl.BlockSpec((Hd * R, C * R), lambda t, n: (0, 0)),
            pl.BlockSpec((Hd * R, 1), lambda t, n: (0, 0)),
            pl.BlockSpec((C * R, Hd * R), lambda t, n: (0, 0)),
            pl.BlockSpec((C * R, 1), lambda t, n: (0, 0)),
        ],
        out_specs=pl.BlockSpec((1, C, RB, W), lambda t, n: (n, 0, t, 0)),
        out_shape=jax.ShapeDtypeStruct((N, C, H, W), jnp.float32),
        compiler_params=pltpu.CompilerParams(
            dimension_semantics=("arbitrary", "arbitrary")),
    )(X, mask2d.astype(jnp.float32), we_k, be_k, wd_k, bd_k)
    return out
